# Initial kernel scaffold; baseline (speedup 1.0000x reference)
import jax
import jax.numpy as jnp
from jax.experimental import pallas as pl


def kernel(x, pos, edge_index, edge_attr, W_e1, b_e1, g_e1, t_e1, W_e2, b_e2, g_e2, t_e2, W_n1, b_n1, g_n1, t_n1, W_n2, b_n2, W_c1, b_c1, W_c2, b_c2):
    raise NotImplementedError("write your pallas kernel here")


# speedup vs baseline: 2.0587x; 2.0587x over previous
import functools

import jax
import jax.numpy as jnp
from jax import lax
from jax.experimental import pallas as pl
from jax.experimental.pallas import tpu as pltpu
from jax.experimental.pallas import tpu_sc as plsc

N, E, F, ED, H = 10000, 320000, 128, 4, 32
RW = 128
NC, NS = 2, 16
NW = NC * NS
CH = 128
EW = -(-E // (NW * CH)) * CH
KCH = EW // CH
E_pad = EW * NW
N_pad = 10240
RPT = N_pad // NS

_HIGH = lax.Precision.HIGHEST


def _silu(v):
    return v * jax.nn.sigmoid(v)


def _ln(v, g, b):
    mu = jnp.mean(v, axis=-1, keepdims=True)
    var = jnp.var(v, axis=-1, keepdims=True)
    return (v - mu) * lax.rsqrt(var + 1e-5) * g + b


def _tables_body(xe_ref, ma_ref, mb_ref, a_ref, b_ref):
    xe = xe_ref[...]
    a_ref[...] = jnp.dot(xe, ma_ref[...], preferred_element_type=jnp.float32,
                         precision=_HIGH)
    b_ref[...] = jnp.dot(xe, mb_ref[...], preferred_element_type=jnp.float32,
                         precision=_HIGH)


def _build_tables(xe, ma, mb):
    blk = 1024
    grid = N_pad // blk
    return pl.pallas_call(
        _tables_body,
        grid=(grid,),
        in_specs=[
            pl.BlockSpec((blk, 136), lambda i: (i, 0)),
            pl.BlockSpec((136, RW), lambda i: (0, 0)),
            pl.BlockSpec((136, RW), lambda i: (0, 0)),
        ],
        out_specs=[
            pl.BlockSpec((blk, RW), lambda i: (i, 0)),
            pl.BlockSpec((blk, RW), lambda i: (i, 0)),
        ],
        out_shape=[
            jax.ShapeDtypeStruct((N_pad, RW), jnp.float32),
            jax.ShapeDtypeStruct((N_pad, RW), jnp.float32),
        ],
    )(xe, ma, mb)


_sc_mesh = plsc.VectorSubcoreMesh(core_axis_name="c", subcore_axis_name="s")


@functools.partial(
    pl.kernel,
    out_type=jax.ShapeDtypeStruct((E_pad, RW), jnp.float32),
    mesh=_sc_mesh,
    scratch_types=[
        pltpu.VMEM((KCH, CH), jnp.int32),
        pltpu.VMEM((KCH, CH), jnp.int32),
        pltpu.VMEM((CH, RW), jnp.float32),
        pltpu.SemaphoreType.DMA,
    ],
)
def _sc_gather(ta, tb, src_i, dst_i, out, src_v, dst_v, buf, sem):
    c = lax.axis_index("c")
    s = lax.axis_index("s")
    wid = s * NC + c
    pltpu.sync_copy(src_i.at[wid], src_v)
    pltpu.sync_copy(dst_i.at[wid], dst_v)

    def body(j, carry):
        pltpu.async_copy(ta.at[src_v.at[j]], buf, sem).wait()
        pltpu.async_copy(tb.at[dst_v.at[j]], buf, sem, add=True).wait()
        pltpu.sync_copy(buf, out.at[pl.ds(wid * EW + j * CH, CH)])
        return carry

    lax.fori_loop(0, KCH, body, 0)


def _edge_body(hrel_ref, ea_ref, weaT_ref, wr2_ref, be1_ref, ge1_ref, te1_ref,
               we2T_ref, be2_ref, ge2_ref, te2_ref, wc1T_ref, bc1_ref,
               wc2T_ref, bc2_ref, out_ref):
    hp = hrel_ref[:, 0:H]
    rel = hrel_ref[:, H:H + 3]
    r2 = jnp.sum(rel * rel, axis=-1, keepdims=True)
    z = (hp
         + jnp.dot(ea_ref[...], weaT_ref[...],
                   preferred_element_type=jnp.float32, precision=_HIGH)
         + r2 * wr2_ref[...]
         + be1_ref[...])
    h = _ln(_silu(z), ge1_ref[...], te1_ref[...])
    m = _ln(_silu(jnp.dot(h, we2T_ref[...], preferred_element_type=jnp.float32,
                          precision=_HIGH) + be2_ref[...]),
            ge2_ref[...], te2_ref[...])
    cc = _silu(jnp.dot(m, wc1T_ref[...], preferred_element_type=jnp.float32,
                       precision=_HIGH) + bc1_ref[...])
    c = jnp.dot(cc, wc2T_ref[...], preferred_element_type=jnp.float32,
                precision=_HIGH) + bc2_ref[...]
    out = jnp.concatenate(
        [m, rel * c, jnp.zeros((m.shape[0], RW - H - 3), jnp.float32)], axis=-1)
    out_ref[...] = out


def _edge_mlp(hrel, ea, weaT, wr2, be1, ge1, te1, we2T, be2, ge2, te2,
              wc1T, bc1, wc2T, bc2):
    blk = 4096
    grid = E_pad // blk
    row = lambda i: (0, 0)
    return pl.pallas_call(
        _edge_body,
        grid=(grid,),
        in_specs=[
            pl.BlockSpec((blk, RW), lambda i: (i, 0)),
            pl.BlockSpec((blk, ED), lambda i: (i, 0)),
            pl.BlockSpec((ED, H), row),
            pl.BlockSpec((1, H), row),
            pl.BlockSpec((1, H), row),
            pl.BlockSpec((1, H), row),
            pl.BlockSpec((1, H), row),
            pl.BlockSpec((H, H), row),
            pl.BlockSpec((1, H), row),
            pl.BlockSpec((1, H), row),
            pl.BlockSpec((1, H), row),
            pl.BlockSpec((H, H), row),
            pl.BlockSpec((1, H), row),
            pl.BlockSpec((H, 1), row),
            pl.BlockSpec((1, 1), row),
        ],
        out_specs=pl.BlockSpec((blk, RW), lambda i: (i, 0)),
        out_shape=jax.ShapeDtypeStruct((E_pad, RW), jnp.float32),
    )(hrel, ea, weaT, wr2, be1, ge1, te1, we2T, be2, ge2, te2, wc1T, bc1,
      wc2T, bc2)


@functools.partial(
    pl.kernel,
    out_type=jax.ShapeDtypeStruct((NC, N_pad, RW), jnp.float32),
    mesh=_sc_mesh,
    scratch_types=[
        pltpu.VMEM((KCH, CH), jnp.int32),
        pltpu.VMEM((CH, RW), jnp.float32),
        pltpu.VMEM((16, RW), jnp.float32),
        pltpu.VMEM_SHARED((N_pad, RW), jnp.float32),
        pltpu.SemaphoreType.DMA,
    ],
)
def _sc_scatter(mtr, dst_i, out, dst_v, buf, zbuf, acc, sem):
    c = lax.axis_index("c")
    s = lax.axis_index("s")
    wid = s * NC + c
    zv = jnp.zeros((16,), jnp.float32)
    for i in range(16):
        for jj in range(RW // 16):
            zbuf[i, pl.ds(jj * 16, 16)] = zv

    def zbody(t, carry):
        pltpu.sync_copy(zbuf, acc.at[pl.ds(s * RPT + t * 16, 16)])
        return carry

    lax.fori_loop(0, RPT // 16, zbody, 0)
    plsc.subcore_barrier()

    pltpu.sync_copy(dst_i.at[wid], dst_v)

    def body(j, carry):
        pltpu.sync_copy(mtr.at[pl.ds(wid * EW + j * CH, CH)], buf)
        pltpu.sync_copy(buf, acc.at[dst_v.at[j]], add=True)
        return carry

    lax.fori_loop(0, KCH, body, 0)
    plsc.subcore_barrier()

    def dbody(t, carry):
        pltpu.sync_copy(acc.at[pl.ds(s * RPT + t * CH, CH)],
                        out.at[c].at[pl.ds(s * RPT + t * CH, CH)])
        return carry

    lax.fori_loop(0, RPT // CH, dbody, 0)


def _node_body(x_ref, pos_ref, p0_ref, p1_ref, wn1xT_ref, wn1aT_ref, bn1_ref,
               gn1_ref, tn1_ref, wn2T_ref, bn2_ref, xo_ref, po_ref):
    x = x_ref[...]
    agg = p0_ref[0, :, 0:H] + p1_ref[0, :, 0:H]
    dpos = p0_ref[0, :, H:H + 3] + p1_ref[0, :, H:H + 3]
    z = (jnp.dot(x, wn1xT_ref[...], preferred_element_type=jnp.float32,
                 precision=_HIGH)
         + jnp.dot(agg, wn1aT_ref[...], preferred_element_type=jnp.float32,
                   precision=_HIGH)
         + bn1_ref[...])
    u = _ln(_silu(z), gn1_ref[...], tn1_ref[...])
    xo_ref[...] = x + jnp.dot(u, wn2T_ref[...],
                              preferred_element_type=jnp.float32,
                              precision=_HIGH) + bn2_ref[...]
    po_ref[...] = pos_ref[...] + dpos


def _node_mlp(x, pos, parts, wn1xT, wn1aT, bn1, gn1, tn1, wn2T, bn2):
    blk = 1000
    grid = N // blk
    row = lambda i: (0, 0)
    return pl.pallas_call(
        _node_body,
        grid=(grid,),
        in_specs=[
            pl.BlockSpec((blk, F), lambda i: (i, 0)),
            pl.BlockSpec((blk, 3), lambda i: (i, 0)),
            pl.BlockSpec((1, blk, RW), lambda i: (0, i, 0)),
            pl.BlockSpec((1, blk, RW), lambda i: (0, i, 0)),
            pl.BlockSpec((F, H), row),
            pl.BlockSpec((H, H), row),
            pl.BlockSpec((1, H), row),
            pl.BlockSpec((1, H), row),
            pl.BlockSpec((1, H), row),
            pl.BlockSpec((H, F), row),
            pl.BlockSpec((1, F), row),
        ],
        out_specs=[
            pl.BlockSpec((blk, F), lambda i: (i, 0)),
            pl.BlockSpec((blk, 3), lambda i: (i, 0)),
        ],
        out_shape=[
            jax.ShapeDtypeStruct((N, F), jnp.float32),
            jax.ShapeDtypeStruct((N, 3), jnp.float32),
        ],
    )(x, pos, parts[0:1], parts[1:2], wn1xT, wn1aT, bn1, gn1, tn1, wn2T, bn2)


def kernel(x, pos, edge_index, edge_attr, W_e1, b_e1, g_e1, t_e1, W_e2, b_e2,
           g_e2, t_e2, W_n1, b_n1, g_n1, t_n1, W_n2, b_n2, W_c1, b_c1, W_c2,
           b_c2):
    f32 = jnp.float32
    wa = W_e1[:, 0:F]
    wb = W_e1[:, F:2 * F]
    weaT = W_e1[:, 2 * F:2 * F + ED].T
    wr2 = W_e1[:, 2 * F + ED:2 * F + ED + 1].T
    eye3 = jnp.eye(3, dtype=f32)
    ma = jnp.zeros((136, RW), f32)
    ma = ma.at[0:F, 0:H].set(wa.T).at[F:F + 3, H:H + 3].set(-eye3)
    mb = jnp.zeros((136, RW), f32)
    mb = mb.at[0:F, 0:H].set(wb.T).at[F:F + 3, H:H + 3].set(eye3)

    xe = jnp.concatenate([x, pos], axis=1)
    xe = jnp.pad(xe, ((0, N_pad - N), (0, 136 - (F + 3))))

    pad_e = E_pad - E
    src = jnp.pad(edge_index[0], (0, pad_e), constant_values=N)
    dst = jnp.pad(edge_index[1], (0, pad_e), constant_values=N)
    src_i = src.reshape(NW, KCH, CH)
    dst_i = dst.reshape(NW, KCH, CH)
    ea = jnp.pad(edge_attr, ((0, pad_e), (0, 0)))

    ta, tb = _build_tables(xe, ma, mb)
    hrel = _sc_gather(ta, tb, src_i, dst_i)
    mtr = _edge_mlp(
        hrel, ea, weaT, wr2,
        b_e1.reshape(1, H), g_e1.reshape(1, H), t_e1.reshape(1, H),
        W_e2.T, b_e2.reshape(1, H), g_e2.reshape(1, H), t_e2.reshape(1, H),
        W_c1.T, b_c1.reshape(1, H), W_c2.T, b_c2.reshape(1, 1))
    parts = _sc_scatter(mtr, dst_i)
    x_new, pos_new = _node_mlp(
        x, pos, parts,
        W_n1[:, 0:F].T, W_n1[:, F:F + H].T, b_n1.reshape(1, H),
        g_n1.reshape(1, H), t_n1.reshape(1, H), W_n2.T, b_n2.reshape(1, F))
    return (x_new, pos_new)


# speedup vs baseline: 2.2985x; 1.1165x over previous
import functools

import jax
import jax.numpy as jnp
from jax import lax
from jax.experimental import pallas as pl
from jax.experimental.pallas import tpu as pltpu
from jax.experimental.pallas import tpu_sc as plsc

N, E, F, ED, H = 10000, 320000, 128, 4, 32
RW = 128
NC, NS = 2, 16
NW = NC * NS
CH = 128
EW = -(-E // (NW * CH)) * CH
KCH = EW // CH
E_pad = EW * NW
N_pad = 10240
RPT = N_pad // NS

_HIGH = lax.Precision.HIGHEST


def _silu(v):
    return v * jax.nn.sigmoid(v)


def _ln(v, g, b):
    mu = jnp.mean(v, axis=-1, keepdims=True)
    var = jnp.var(v, axis=-1, keepdims=True)
    return (v - mu) * lax.rsqrt(var + 1e-5) * g + b


def _tables_body(xe_ref, ma_ref, mb_ref, a_ref, b_ref):
    xe = xe_ref[...]
    a_ref[...] = jnp.dot(xe, ma_ref[...], preferred_element_type=jnp.float32,
                         precision=_HIGH)
    b_ref[...] = jnp.dot(xe, mb_ref[...], preferred_element_type=jnp.float32,
                         precision=_HIGH)


def _build_tables(xe, ma, mb):
    blk = 1024
    grid = N_pad // blk
    return pl.pallas_call(
        _tables_body,
        grid=(grid,),
        in_specs=[
            pl.BlockSpec((blk, 136), lambda i: (i, 0)),
            pl.BlockSpec((136, RW), lambda i: (0, 0)),
            pl.BlockSpec((136, RW), lambda i: (0, 0)),
        ],
        out_specs=[
            pl.BlockSpec((blk, RW), lambda i: (i, 0)),
            pl.BlockSpec((blk, RW), lambda i: (i, 0)),
        ],
        out_shape=[
            jax.ShapeDtypeStruct((N_pad, RW), jnp.float32),
            jax.ShapeDtypeStruct((N_pad, RW), jnp.float32),
        ],
    )(xe, ma, mb)


_sc_mesh = plsc.VectorSubcoreMesh(core_axis_name="c", subcore_axis_name="s")


NBUF = 4


@functools.partial(
    pl.kernel,
    out_type=jax.ShapeDtypeStruct((E_pad, RW), jnp.float32),
    mesh=_sc_mesh,
    scratch_types=[
        pltpu.VMEM((KCH, CH), jnp.int32),
        pltpu.VMEM((KCH, CH), jnp.int32),
    ] + [pltpu.VMEM((CH, RW), jnp.float32) for _ in range(NBUF)]
      + [pltpu.SemaphoreType.DMA for _ in range(NBUF)],
)
def _sc_gather(ta, tb, src_i, dst_i, out, src_v, dst_v, b0, b1, b2, b3,
               s0, s1, s2, s3):
    bufs = (b0, b1, b2, b3)
    sems = (s0, s1, s2, s3)
    c = lax.axis_index("c")
    s = lax.axis_index("s")
    wid = s * NC + c
    pltpu.sync_copy(src_i.at[wid], src_v)
    pltpu.sync_copy(dst_i.at[wid], dst_v)

    def start_a(j, b):
        pltpu.async_copy(ta.at[src_v.at[j]], bufs[b], sems[b])

    def wait_ab(b):
        pltpu.make_async_copy(ta.at[src_v.at[0]], bufs[b], sems[b]).wait()

    def start_b(j, b):
        pltpu.async_copy(tb.at[dst_v.at[j]], bufs[b], sems[b], add=True)

    def start_o(j, b):
        pltpu.async_copy(bufs[b], out.at[pl.ds(wid * EW + j * CH, CH)],
                         sems[b])

    def wait_o(j, b):
        pltpu.make_async_copy(
            bufs[b], out.at[pl.ds(wid * EW + j * CH, CH)], sems[b]).wait()

    start_a(0, 0)
    start_a(1, 1)
    start_a(2, 2)
    wait_ab(0)
    start_b(0, 0)

    def outer(g, carry):
        for b0_ in range(NBUF):
            j = g * NBUF + b0_
            bb = b0_
            nb = (b0_ + 1) % NBUF
            ab = (b0_ + 3) % NBUF

            @pl.when(j < KCH)
            def _():
                wait_ab(bb)
                start_o(j, bb)

            @pl.when(j + 1 < KCH)
            def _():
                wait_ab(nb)
                start_b(j + 1, nb)

            @pl.when(j == 0)
            def _():
                start_a(3, 3)

            @pl.when((j >= 1) & (j + 3 < KCH))
            def _():
                wait_o(j - 1, ab)
                start_a(j + 3, ab)

        return carry

    lax.fori_loop(0, (KCH + NBUF - 1) // NBUF, outer, 0)
    for x in range(max(0, KCH - 4), KCH):
        wait_o(x, x % NBUF)


def _edge_body(hrel_ref, ea_ref, weaT_ref, wr2_ref, be1_ref, ge1_ref, te1_ref,
               we2T_ref, be2_ref, ge2_ref, te2_ref, wc1T_ref, bc1_ref,
               wc2T_ref, bc2_ref, out_ref):
    hp = hrel_ref[:, 0:H]
    rel = hrel_ref[:, H:H + 3]
    r2 = jnp.sum(rel * rel, axis=-1, keepdims=True)
    z = (hp
         + jnp.dot(ea_ref[...], weaT_ref[...],
                   preferred_element_type=jnp.float32, precision=_HIGH)
         + r2 * wr2_ref[...]
         + be1_ref[...])
    h = _ln(_silu(z), ge1_ref[...], te1_ref[...])
    m = _ln(_silu(jnp.dot(h, we2T_ref[...], preferred_element_type=jnp.float32,
                          precision=_HIGH) + be2_ref[...]),
            ge2_ref[...], te2_ref[...])
    cc = _silu(jnp.dot(m, wc1T_ref[...], preferred_element_type=jnp.float32,
                       precision=_HIGH) + bc1_ref[...])
    c = jnp.dot(cc, wc2T_ref[...], preferred_element_type=jnp.float32,
                precision=_HIGH) + bc2_ref[...]
    out = jnp.concatenate(
        [m, rel * c, jnp.zeros((m.shape[0], RW - H - 3), jnp.float32)], axis=-1)
    out_ref[...] = out


def _edge_mlp(hrel, ea, weaT, wr2, be1, ge1, te1, we2T, be2, ge2, te2,
              wc1T, bc1, wc2T, bc2):
    blk = 4096
    grid = E_pad // blk
    row = lambda i: (0, 0)
    return pl.pallas_call(
        _edge_body,
        grid=(grid,),
        in_specs=[
            pl.BlockSpec((blk, RW), lambda i: (i, 0)),
            pl.BlockSpec((blk, ED), lambda i: (i, 0)),
            pl.BlockSpec((ED, H), row),
            pl.BlockSpec((1, H), row),
            pl.BlockSpec((1, H), row),
            pl.BlockSpec((1, H), row),
            pl.BlockSpec((1, H), row),
            pl.BlockSpec((H, H), row),
            pl.BlockSpec((1, H), row),
            pl.BlockSpec((1, H), row),
            pl.BlockSpec((1, H), row),
            pl.BlockSpec((H, H), row),
            pl.BlockSpec((1, H), row),
            pl.BlockSpec((H, 1), row),
            pl.BlockSpec((1, 1), row),
        ],
        out_specs=pl.BlockSpec((blk, RW), lambda i: (i, 0)),
        out_shape=jax.ShapeDtypeStruct((E_pad, RW), jnp.float32),
    )(hrel, ea, weaT, wr2, be1, ge1, te1, we2T, be2, ge2, te2, wc1T, bc1,
      wc2T, bc2)


@functools.partial(
    pl.kernel,
    out_type=jax.ShapeDtypeStruct((NC, N_pad, RW), jnp.float32),
    mesh=_sc_mesh,
    scratch_types=[
        pltpu.VMEM((KCH, CH), jnp.int32),
        pltpu.VMEM((16, RW), jnp.float32),
        pltpu.VMEM_SHARED((N_pad, RW), jnp.float32),
    ] + [pltpu.VMEM((CH, RW), jnp.float32) for _ in range(2)]
      + [pltpu.SemaphoreType.DMA for _ in range(2)],
)
def _sc_scatter(mtr, dst_i, out, dst_v, zbuf, acc, b0, b1, s0, s1):
    bufs = (b0, b1)
    sems = (s0, s1)
    NB = 2
    c = lax.axis_index("c")
    s = lax.axis_index("s")
    wid = s * NC + c
    zv = jnp.zeros((16,), jnp.float32)
    for i in range(16):
        for jj in range(RW // 16):
            zbuf[i, pl.ds(jj * 16, 16)] = zv

    def zbody(t, carry):
        pltpu.sync_copy(zbuf, acc.at[pl.ds(s * RPT + t * 16, 16)])
        return carry

    lax.fori_loop(0, RPT // 16, zbody, 0)
    plsc.subcore_barrier()

    pltpu.sync_copy(dst_i.at[wid], dst_v)

    def start_l(j, b):
        pltpu.async_copy(mtr.at[pl.ds(wid * EW + j * CH, CH)], bufs[b],
                         sems[b])

    def wait_l(b):
        pltpu.make_async_copy(mtr.at[pl.ds(wid * EW, CH)], bufs[b],
                              sems[b]).wait()

    def start_s(j, b):
        pltpu.async_copy(bufs[b], acc.at[dst_v.at[j]], sems[b], add=True)

    def wait_s(j, b):
        pltpu.make_async_copy(bufs[b], acc.at[dst_v.at[j]], sems[b]).wait()

    start_l(0, 0)

    def outer(g, carry):
        for b0_ in range(NB):
            j = g * NB + b0_
            bb = b0_
            nb = (b0_ + 1) % NB

            @pl.when(j < KCH)
            def _():
                wait_l(bb)
                start_s(j, bb)

            @pl.when((j >= 1) & (j + 1 < KCH))
            def _():
                wait_s(j - 1, nb)

            @pl.when(j + 1 < KCH)
            def _():
                start_l(j + 1, nb)

        return carry

    lax.fori_loop(0, (KCH + NB - 1) // NB, outer, 0)
    for x in range(max(0, KCH - 2), KCH):
        wait_s(x, x % NB)
    plsc.subcore_barrier()

    def dbody(t, carry):
        pltpu.sync_copy(acc.at[pl.ds(s * RPT + t * CH, CH)],
                        out.at[c].at[pl.ds(s * RPT + t * CH, CH)])
        return carry

    lax.fori_loop(0, RPT // CH, dbody, 0)


def _node_body(x_ref, pos_ref, p0_ref, p1_ref, wn1xT_ref, wn1aT_ref, bn1_ref,
               gn1_ref, tn1_ref, wn2T_ref, bn2_ref, xo_ref, po_ref):
    x = x_ref[...]
    agg = p0_ref[0, :, 0:H] + p1_ref[0, :, 0:H]
    dpos = p0_ref[0, :, H:H + 3] + p1_ref[0, :, H:H + 3]
    z = (jnp.dot(x, wn1xT_ref[...], preferred_element_type=jnp.float32,
                 precision=_HIGH)
         + jnp.dot(agg, wn1aT_ref[...], preferred_element_type=jnp.float32,
                   precision=_HIGH)
         + bn1_ref[...])
    u = _ln(_silu(z), gn1_ref[...], tn1_ref[...])
    xo_ref[...] = x + jnp.dot(u, wn2T_ref[...],
                              preferred_element_type=jnp.float32,
                              precision=_HIGH) + bn2_ref[...]
    po_ref[...] = pos_ref[...] + dpos


def _node_mlp(x, pos, parts, wn1xT, wn1aT, bn1, gn1, tn1, wn2T, bn2):
    blk = 1000
    grid = N // blk
    row = lambda i: (0, 0)
    return pl.pallas_call(
        _node_body,
        grid=(grid,),
        in_specs=[
            pl.BlockSpec((blk, F), lambda i: (i, 0)),
            pl.BlockSpec((blk, 3), lambda i: (i, 0)),
            pl.BlockSpec((1, blk, RW), lambda i: (0, i, 0)),
            pl.BlockSpec((1, blk, RW), lambda i: (0, i, 0)),
            pl.BlockSpec((F, H), row),
            pl.BlockSpec((H, H), row),
            pl.BlockSpec((1, H), row),
            pl.BlockSpec((1, H), row),
            pl.BlockSpec((1, H), row),
            pl.BlockSpec((H, F), row),
            pl.BlockSpec((1, F), row),
        ],
        out_specs=[
            pl.BlockSpec((blk, F), lambda i: (i, 0)),
            pl.BlockSpec((blk, 3), lambda i: (i, 0)),
        ],
        out_shape=[
            jax.ShapeDtypeStruct((N, F), jnp.float32),
            jax.ShapeDtypeStruct((N, 3), jnp.float32),
        ],
    )(x, pos, parts[0:1], parts[1:2], wn1xT, wn1aT, bn1, gn1, tn1, wn2T, bn2)


def kernel(x, pos, edge_index, edge_attr, W_e1, b_e1, g_e1, t_e1, W_e2, b_e2,
           g_e2, t_e2, W_n1, b_n1, g_n1, t_n1, W_n2, b_n2, W_c1, b_c1, W_c2,
           b_c2):
    f32 = jnp.float32
    wa = W_e1[:, 0:F]
    wb = W_e1[:, F:2 * F]
    weaT = W_e1[:, 2 * F:2 * F + ED].T
    wr2 = W_e1[:, 2 * F + ED:2 * F + ED + 1].T
    eye3 = jnp.eye(3, dtype=f32)
    ma = jnp.zeros((136, RW), f32)
    ma = ma.at[0:F, 0:H].set(wa.T).at[F:F + 3, H:H + 3].set(-eye3)
    mb = jnp.zeros((136, RW), f32)
    mb = mb.at[0:F, 0:H].set(wb.T).at[F:F + 3, H:H + 3].set(eye3)

    xe = jnp.concatenate([x, pos], axis=1)
    xe = jnp.pad(xe, ((0, N_pad - N), (0, 136 - (F + 3))))

    pad_e = E_pad - E
    src = jnp.pad(edge_index[0], (0, pad_e), constant_values=N)
    dst = jnp.pad(edge_index[1], (0, pad_e), constant_values=N)
    src_i = src.reshape(NW, KCH, CH)
    dst_i = dst.reshape(NW, KCH, CH)
    ea = jnp.pad(edge_attr, ((0, pad_e), (0, 0)))

    ta, tb = _build_tables(xe, ma, mb)
    hrel = _sc_gather(ta, tb, src_i, dst_i)
    mtr = _edge_mlp(
        hrel, ea, weaT, wr2,
        b_e1.reshape(1, H), g_e1.reshape(1, H), t_e1.reshape(1, H),
        W_e2.T, b_e2.reshape(1, H), g_e2.reshape(1, H), t_e2.reshape(1, H),
        W_c1.T, b_c1.reshape(1, H), W_c2.T, b_c2.reshape(1, 1))
    parts = _sc_scatter(mtr, dst_i)
    x_new, pos_new = _node_mlp(
        x, pos, parts,
        W_n1[:, 0:F].T, W_n1[:, F:F + H].T, b_n1.reshape(1, H),
        g_n1.reshape(1, H), t_n1.reshape(1, H), W_n2.T, b_n2.reshape(1, F))
    return (x_new, pos_new)


# speedup vs baseline: 4.5609x; 1.9843x over previous
import functools

import jax
import jax.numpy as jnp
from jax import lax
from jax.experimental import pallas as pl
from jax.experimental.pallas import tpu as pltpu
from jax.experimental.pallas import tpu_sc as plsc

N, E, F, ED, H = 10000, 320000, 128, 4, 32
RW = 128
NC, NS = 2, 16
NW = NC * NS
CH = 128
EW = -(-E // (NW * CH)) * CH
KCH = EW // CH
E_pad = EW * NW
N_pad = 10240
RPT = N_pad // NS

_HIGH = lax.Precision.HIGHEST


def _silu(v):
    return v * jax.nn.sigmoid(v)


def _ln(v, g, b):
    mu = jnp.mean(v, axis=-1, keepdims=True)
    var = jnp.var(v, axis=-1, keepdims=True)
    return (v - mu) * lax.rsqrt(var + 1e-5) * g + b


def _tables_body(xe_ref, ma_ref, mb_ref, a_ref, b_ref):
    xe = xe_ref[...]
    a_ref[...] = jnp.dot(xe, ma_ref[...], preferred_element_type=jnp.float32,
                         precision=_HIGH)
    b_ref[...] = jnp.dot(xe, mb_ref[...], preferred_element_type=jnp.float32,
                         precision=_HIGH)


def _build_tables(xe, ma, mb):
    blk = 1024
    grid = N_pad // blk
    return pl.pallas_call(
        _tables_body,
        grid=(grid,),
        in_specs=[
            pl.BlockSpec((blk, 136), lambda i: (i, 0)),
            pl.BlockSpec((136, RW), lambda i: (0, 0)),
            pl.BlockSpec((136, RW), lambda i: (0, 0)),
        ],
        out_specs=[
            pl.BlockSpec((blk, RW), lambda i: (i, 0)),
            pl.BlockSpec((blk, RW), lambda i: (i, 0)),
        ],
        out_shape=[
            jax.ShapeDtypeStruct((N_pad, RW), jnp.float32),
            jax.ShapeDtypeStruct((N_pad, RW), jnp.float32),
        ],
    )(xe, ma, mb)


_sc_mesh = plsc.VectorSubcoreMesh(core_axis_name="c", subcore_axis_name="s")


NBUF = 4


def _make_sc_gather(KH):
    EWh = KH * CH

    @functools.partial(
        pl.kernel,
        out_type=jax.ShapeDtypeStruct((NW * EWh, RW), jnp.float32),
        mesh=_sc_mesh,
        scratch_types=[
            pltpu.VMEM((KH, CH), jnp.int32),
            pltpu.VMEM((KH, CH), jnp.int32),
        ] + [pltpu.VMEM((CH, RW), jnp.float32) for _ in range(NBUF)]
          + [pltpu.SemaphoreType.DMA for _ in range(NBUF)],
    )
    def _sc_gather(ta, tb, src_i, dst_i, out, src_v, dst_v, b0, b1, b2, b3,
                   s0, s1, s2, s3):
        bufs = (b0, b1, b2, b3)
        sems = (s0, s1, s2, s3)
        c = lax.axis_index("c")
        s = lax.axis_index("s")
        wid = s * NC + c
        pltpu.sync_copy(src_i.at[wid], src_v)
        pltpu.sync_copy(dst_i.at[wid], dst_v)

        def start_a(j, b):
            pltpu.async_copy(ta.at[src_v.at[j]], bufs[b], sems[b])

        def wait_ab(b):
            pltpu.make_async_copy(ta.at[src_v.at[0]], bufs[b],
                                  sems[b]).wait()

        def start_b(j, b):
            pltpu.async_copy(tb.at[dst_v.at[j]], bufs[b], sems[b], add=True)

        def start_o(j, b):
            pltpu.async_copy(bufs[b], out.at[pl.ds(wid * EWh + j * CH, CH)],
                             sems[b])

        def wait_o(j, b):
            pltpu.make_async_copy(
                bufs[b], out.at[pl.ds(wid * EWh + j * CH, CH)],
                sems[b]).wait()

        start_a(0, 0)
        start_a(1, 1)
        start_a(2, 2)
        wait_ab(0)
        start_b(0, 0)

        def outer(g, carry):
            for b0_ in range(NBUF):
                j = g * NBUF + b0_
                bb = b0_
                nb = (b0_ + 1) % NBUF
                ab = (b0_ + 3) % NBUF

                @pl.when(j < KH)
                def _():
                    wait_ab(bb)
                    start_o(j, bb)

                @pl.when(j + 1 < KH)
                def _():
                    wait_ab(nb)
                    start_b(j + 1, nb)

                @pl.when(j == 0)
                def _():
                    start_a(3, 3)

                @pl.when((j >= 1) & (j + 3 < KH))
                def _():
                    wait_o(j - 1, ab)
                    start_a(j + 3, ab)

            return carry

        lax.fori_loop(0, (KH + NBUF - 1) // NBUF, outer, 0)
        for x in range(max(0, KH - 4), KH):
            wait_o(x, x % NBUF)

    return _sc_gather


_BE = 4096


def _edge_body(hrel_ref, ea_ref, wea_ref, wr2c_ref, be1c_ref, ge1c_ref,
               te1c_ref, we2_ref, be2c_ref, ge2c_ref, te2c_ref, wc1_ref,
               bc1c_ref, wc2_ref, bc2c_ref, out_ref):
    def dotp(a, b):
        return jnp.dot(a, b, preferred_element_type=jnp.float32,
                       precision=_HIGH)

    def ln_t(v, g, t):
        mu = jnp.mean(v, axis=0, keepdims=True)
        var = jnp.mean(v * v, axis=0, keepdims=True) - mu * mu
        return (v - mu) * lax.rsqrt(var + 1e-5) * g + t

    hp = hrel_ref[:, 0:H].T
    rel = hrel_ref[:, H:H + 3].T
    ea = ea_ref[...].T
    r2 = jnp.sum(rel * rel, axis=0, keepdims=True)
    z = (hp + dotp(wea_ref[...], ea) + wr2c_ref[...] * r2 + be1c_ref[...])
    h = ln_t(_silu(z), ge1c_ref[...], te1c_ref[...])
    m = ln_t(_silu(dotp(we2_ref[...], h) + be2c_ref[...]),
             ge2c_ref[...], te2c_ref[...])
    cc = _silu(dotp(wc1_ref[...], m) + bc1c_ref[...])
    c = dotp(wc2_ref[...], cc) + bc2c_ref[...]
    trans = rel * c
    out = jnp.concatenate(
        [m.T, trans.T, jnp.zeros((_BE, RW - H - 3), jnp.float32)], axis=-1)
    out_ref[...] = out


def _edge_mlp(hrel, ea, wea, wr2c, be1c, ge1c, te1c, we2, be2c, ge2c, te2c,
              wc1, bc1c, wc2, bc2c):
    grid = hrel.shape[0] // _BE
    row = lambda i: (0, 0)
    return pl.pallas_call(
        _edge_body,
        grid=(grid,),
        in_specs=[
            pl.BlockSpec((_BE, RW), lambda i: (i, 0)),
            pl.BlockSpec((_BE, ED), lambda i: (i, 0)),
            pl.BlockSpec((H, ED), row),
            pl.BlockSpec((H, 1), row),
            pl.BlockSpec((H, 1), row),
            pl.BlockSpec((H, 1), row),
            pl.BlockSpec((H, 1), row),
            pl.BlockSpec((H, H), row),
            pl.BlockSpec((H, 1), row),
            pl.BlockSpec((H, 1), row),
            pl.BlockSpec((H, 1), row),
            pl.BlockSpec((H, H), row),
            pl.BlockSpec((H, 1), row),
            pl.BlockSpec((1, H), row),
            pl.BlockSpec((1, 1), row),
        ],
        out_specs=pl.BlockSpec((_BE, RW), lambda i: (i, 0)),
        out_shape=jax.ShapeDtypeStruct((hrel.shape[0], RW), jnp.float32),
    )(hrel, ea, wea, wr2c, be1c, ge1c, te1c, we2, be2c, ge2c, te2c,
      wc1, bc1c, wc2, bc2c)


def _make_sc_scatter(KH):
    EWh = KH * CH

    @functools.partial(
        pl.kernel,
        out_type=jax.ShapeDtypeStruct((NC, N_pad, RW), jnp.float32),
        mesh=_sc_mesh,
        scratch_types=[
            pltpu.VMEM((KH, CH), jnp.int32),
            pltpu.VMEM((16, RW), jnp.float32),
            pltpu.VMEM_SHARED((N_pad, RW), jnp.float32),
        ] + [pltpu.VMEM((CH, RW), jnp.float32) for _ in range(2)]
          + [pltpu.SemaphoreType.DMA for _ in range(2)],
    )
    def _sc_scatter(mtr, dst_i, out, dst_v, zbuf, acc, b0, b1, s0, s1):
        bufs = (b0, b1)
        sems = (s0, s1)
        NB = 2
        c = lax.axis_index("c")
        s = lax.axis_index("s")
        wid = s * NC + c
        zv = jnp.zeros((16,), jnp.float32)
        for i in range(16):
            for jj in range(RW // 16):
                zbuf[i, pl.ds(jj * 16, 16)] = zv

        def zbody(t, carry):
            pltpu.sync_copy(zbuf, acc.at[pl.ds(s * RPT + t * 16, 16)])
            return carry

        lax.fori_loop(0, RPT // 16, zbody, 0)
        plsc.subcore_barrier()

        pltpu.sync_copy(dst_i.at[wid], dst_v)

        def start_l(j, b):
            pltpu.async_copy(mtr.at[pl.ds(wid * EWh + j * CH, CH)], bufs[b],
                             sems[b])

        def wait_l(b):
            pltpu.make_async_copy(mtr.at[pl.ds(wid * EWh, CH)], bufs[b],
                                  sems[b]).wait()

        def start_s(j, b):
            pltpu.async_copy(bufs[b], acc.at[dst_v.at[j]], sems[b], add=True)

        def wait_s(j, b):
            pltpu.make_async_copy(bufs[b], acc.at[dst_v.at[j]],
                                  sems[b]).wait()

        start_l(0, 0)

        def outer(g, carry):
            for b0_ in range(NB):
                j = g * NB + b0_
                bb = b0_
                nb = (b0_ + 1) % NB

                @pl.when(j < KH)
                def _():
                    wait_l(bb)
                    start_s(j, bb)

                @pl.when((j >= 1) & (j + 1 < KH))
                def _():
                    wait_s(j - 1, nb)

                @pl.when(j + 1 < KH)
                def _():
                    start_l(j + 1, nb)

            return carry

        lax.fori_loop(0, (KH + NB - 1) // NB, outer, 0)
        for x in range(max(0, KH - 2), KH):
            wait_s(x, x % NB)
        plsc.subcore_barrier()

        def dbody(t, carry):
            pltpu.sync_copy(acc.at[pl.ds(s * RPT + t * CH, CH)],
                            out.at[c].at[pl.ds(s * RPT + t * CH, CH)])
            return carry

        lax.fori_loop(0, RPT // CH, dbody, 0)

    return _sc_scatter


HALVES = ((0, KCH),)
_KHS = sorted({b - a for a, b in HALVES})
_GATHERS = {k: _make_sc_gather(k) for k in _KHS}
_SCATTERS = {k: _make_sc_scatter(k) for k in _KHS}


def _node_mlp(x, pos, parts_list, wn1xT, wn1aT, bn1, gn1, tn1, wn2T, bn2):
    blk = 1000
    grid = N // blk
    row = lambda i: (0, 0)
    nparts = len(parts_list)

    def body(*refs):
        x_ref, pos_ref = refs[0], refs[1]
        p_refs = refs[2:2 + nparts]
        (wn1xT_ref, wn1aT_ref, bn1_ref, gn1_ref, tn1_ref, wn2T_ref,
         bn2_ref) = refs[2 + nparts:2 + nparts + 7]
        xo_ref, po_ref = refs[-2], refs[-1]
        x_ = x_ref[...]
        agg = p_refs[0][0, :, 0:H] + p_refs[0][1, :, 0:H]
        dpos = p_refs[0][0, :, H:H + 3] + p_refs[0][1, :, H:H + 3]
        for p in p_refs[1:]:
            agg = agg + p[0, :, 0:H] + p[1, :, 0:H]
            dpos = dpos + p[0, :, H:H + 3] + p[1, :, H:H + 3]
        z = (jnp.dot(x_, wn1xT_ref[...], preferred_element_type=jnp.float32,
                     precision=_HIGH)
             + jnp.dot(agg, wn1aT_ref[...],
                       preferred_element_type=jnp.float32, precision=_HIGH)
             + bn1_ref[...])
        u = _ln(_silu(z), gn1_ref[...], tn1_ref[...])
        xo_ref[...] = x_ + jnp.dot(u, wn2T_ref[...],
                                   preferred_element_type=jnp.float32,
                                   precision=_HIGH) + bn2_ref[...]
        po_ref[...] = pos_ref[...] + dpos

    return pl.pallas_call(
        body,
        grid=(grid,),
        in_specs=[
            pl.BlockSpec((blk, F), lambda i: (i, 0)),
            pl.BlockSpec((blk, 3), lambda i: (i, 0)),
        ] + [pl.BlockSpec((NC, blk, RW), lambda i: (0, i, 0))
             for _ in range(nparts)] + [
            pl.BlockSpec((F, H), row),
            pl.BlockSpec((H, H), row),
            pl.BlockSpec((1, H), row),
            pl.BlockSpec((1, H), row),
            pl.BlockSpec((1, H), row),
            pl.BlockSpec((H, F), row),
            pl.BlockSpec((1, F), row),
        ],
        out_specs=[
            pl.BlockSpec((blk, F), lambda i: (i, 0)),
            pl.BlockSpec((blk, 3), lambda i: (i, 0)),
        ],
        out_shape=[
            jax.ShapeDtypeStruct((N, F), jnp.float32),
            jax.ShapeDtypeStruct((N, 3), jnp.float32),
        ],
    )(x, pos, *parts_list, wn1xT, wn1aT, bn1, gn1, tn1, wn2T, bn2)


def kernel(x, pos, edge_index, edge_attr, W_e1, b_e1, g_e1, t_e1, W_e2, b_e2,
           g_e2, t_e2, W_n1, b_n1, g_n1, t_n1, W_n2, b_n2, W_c1, b_c1, W_c2,
           b_c2):
    f32 = jnp.float32
    wa = W_e1[:, 0:F]
    wb = W_e1[:, F:2 * F]
    weaT = W_e1[:, 2 * F:2 * F + ED].T
    wr2 = W_e1[:, 2 * F + ED:2 * F + ED + 1].T
    eye3 = jnp.eye(3, dtype=f32)
    ma = jnp.zeros((136, RW), f32)
    ma = ma.at[0:F, 0:H].set(wa.T).at[F:F + 3, H:H + 3].set(-eye3)
    mb = jnp.zeros((136, RW), f32)
    mb = mb.at[0:F, 0:H].set(wb.T).at[F:F + 3, H:H + 3].set(eye3)

    xe = jnp.concatenate([x, pos], axis=1)
    xe = jnp.pad(xe, ((0, N_pad - N), (0, 136 - (F + 3))))

    pad_e = E_pad - E
    src = jnp.pad(edge_index[0], (0, pad_e), constant_values=N)
    dst = jnp.pad(edge_index[1], (0, pad_e), constant_values=N)
    src3 = src.reshape(NW, KCH, CH)
    dst3 = dst.reshape(NW, KCH, CH)
    ea4 = jnp.pad(edge_attr, ((0, pad_e), (0, 0))).reshape(NW, KCH, CH, ED)

    wea = W_e1[:, 2 * F:2 * F + ED]
    wr2c = W_e1[:, 2 * F + ED:2 * F + ED + 1]
    col = lambda v: v.reshape(H, 1)

    ta, tb = _build_tables(xe, ma, mb)
    parts_list = []
    for a, b in HALVES:
        kh = b - a
        hrel_h = _GATHERS[kh](ta, tb, src3[:, a:b], dst3[:, a:b])
        mtr_h = _edge_mlp(
            hrel_h, ea4[:, a:b].reshape(-1, ED), wea, wr2c,
            col(b_e1), col(g_e1), col(t_e1), W_e2, col(b_e2), col(g_e2),
            col(t_e2), W_c1, col(b_c1), W_c2, b_c2.reshape(1, 1))
        parts_list.append(_SCATTERS[kh](mtr_h, dst3[:, a:b]))
    x_new, pos_new = _node_mlp(
        x, pos, parts_list,
        W_n1[:, 0:F].T, W_n1[:, F:F + H].T, b_n1.reshape(1, H),
        g_n1.reshape(1, H), t_n1.reshape(1, H), W_n2.T, b_n2.reshape(1, F))
    return (x_new, pos_new)
